# baseline (speedup 1.0000x reference)
import jax
import jax.numpy as jnp
from jax import lax
from jax.experimental import pallas as pl
from jax.experimental.pallas import tpu as pltpu

_N = 16384
_R = 128
_L = 128


def _partner(a, bset, t, axis):
    return jnp.where(bset, jnp.roll(a, t, axis), jnp.roll(a, -t, axis))


def _cox_body(d_ref, lh_ref, e_ref, out_ref):
    u = lax.bitcast_convert_type(d_ref[...], jnp.int32)
    lh = lh_ref[...]
    e = e_ref[...]
    ri = lax.broadcasted_iota(jnp.int32, (_R, _L), 0)
    ci = lax.broadcasted_iota(jnp.int32, (_R, _L), 1)
    lin = ri * _L + ci
    k1 = -((u & jnp.int32(-16384)) | (jnp.int32(_N - 1) - lin))

    mx = jnp.max(lh)
    w = jnp.exp(lh - mx)
    elh = jnp.sum(e * lh)
    den = jnp.sum(e)
    wb = w.astype(jnp.bfloat16).astype(jnp.float32)
    eb = e.astype(jnp.bfloat16).astype(jnp.float32)
    p = lax.bitcast_convert_type(wb, jnp.int32) | (
        lax.bitcast_convert_type(eb, jnp.int32) >> 16)

    rbits = [(ri & (1 << b)) != 0 for b in range(7)]
    cbits = [(ci & (1 << b)) != 0 for b in range(7)]

    def dmask(k):
        b = k.bit_length() - 1
        return (lin << (31 - b)) >> 31

    K = k1 ^ dmask(2)
    prev_k = 2
    pend = None
    k = 2
    while k <= _N:
        if k != prev_k:
            K = K ^ (dmask(prev_k) ^ dmask(k))
            prev_k = k
        j = k // 2
        while j > 0:
            if j >= _L:
                t, axis = j // _L, 0
                bset = rbits[(j // _L).bit_length() - 1]
            else:
                t, axis = j, 1
                bset = cbits[j.bit_length() - 1]
            pK = _partner(K, bset, t, axis)
            sel = bset != (K > pK)
            K = jnp.where(sel, pK, K)
            if pend is not None:
                psel, pbset, ptt, pax = pend
                p = jnp.where(psel, _partner(p, pbset, ptt, pax), p)
            pend = (sel, bset, t, axis)
            j //= 2
        k *= 2
    psel, pbset, ptt, pax = pend
    p = jnp.where(psel, _partner(p, pbset, ptt, pax), p)

    w_s = lax.bitcast_convert_type(p & jnp.int32(-65536), jnp.float32)
    e_s = lax.bitcast_convert_type(p << 16, jnp.float32)

    ps = w_s
    s = 1
    while s < _L:
        ps = ps + jnp.where(ci >= s, jnp.roll(ps, s, 1), 0.0)
        s *= 2
    rs = ps[:, _L - 1:_L]
    rio = lax.broadcasted_iota(jnp.int32, (_R, 1), 0)
    ro = rs
    s = 1
    while s < _R:
        ro = ro + jnp.where(rio >= s, jnp.roll(ro, s, 0), 0.0)
        s *= 2
    prefix = ps + (ro - rs)
    risk = mx + jnp.log(prefix)
    num = jnp.sum(e_s * risk) - elh
    out_ref[0] = num / den


@jax.jit
def kernel(log_h, y_gts):
    d = y_gts[:, 0]
    e = y_gts[:, 1]
    out = pl.pallas_call(
        _cox_body,
        out_shape=jax.ShapeDtypeStruct((1,), jnp.float32),
        out_specs=pl.BlockSpec(memory_space=pltpu.SMEM),
    )(d.reshape(_R, _L), log_h.reshape(_R, _L), e.reshape(_R, _L))
    return out[0]


# speedup vs baseline: 1.0242x; 1.0242x over previous
import jax
import jax.numpy as jnp
from jax import lax
from jax.experimental import pallas as pl

_N = 16384
_R = 128
_L = 128


def _partner(a, bset, t, axis):
    return jnp.where(bset, jnp.roll(a, t, axis), jnp.roll(a, -t, axis))


def _cox_body(d_ref, lh_ref, e_ref, out_ref):
    u = lax.bitcast_convert_type(d_ref[...], jnp.int32)
    lh = lh_ref[...]
    e = e_ref[...]
    ri = lax.broadcasted_iota(jnp.int32, (_R, _L), 0)
    ci = lax.broadcasted_iota(jnp.int32, (_R, _L), 1)
    lin = ri * _L + ci
    k1 = -((u & jnp.int32(-16384)) | (jnp.int32(_N - 1) - lin))

    mx = jnp.max(lh)
    w = jnp.exp(lh - mx)
    elh = jnp.sum(e * lh)
    den = jnp.sum(e)
    wb = w.astype(jnp.bfloat16).astype(jnp.float32)
    eb = e.astype(jnp.bfloat16).astype(jnp.float32)
    p = lax.bitcast_convert_type(wb, jnp.int32) | (
        lax.bitcast_convert_type(eb, jnp.int32) >> 16)

    rbits = [(ri & (1 << b)) != 0 for b in range(7)]
    cbits = [(ci & (1 << b)) != 0 for b in range(7)]

    def dmask(k):
        b = k.bit_length() - 1
        return (lin << (31 - b)) >> 31

    K = k1 ^ dmask(2)
    prev_k = 2
    pend = None
    k = 2
    while k <= _N:
        if k != prev_k:
            K = K ^ (dmask(prev_k) ^ dmask(k))
            prev_k = k
        j = k // 2
        while j > 0:
            if j >= _L:
                t, axis = j // _L, 0
                bset = rbits[(j // _L).bit_length() - 1]
            else:
                t, axis = j, 1
                bset = cbits[j.bit_length() - 1]
            pK = _partner(K, bset, t, axis)
            sel = bset != (K > pK)
            K = jnp.where(sel, pK, K)
            if pend is not None:
                psel, pbset, ptt, pax = pend
                p = jnp.where(psel, _partner(p, pbset, ptt, pax), p)
            pend = (sel, bset, t, axis)
            j //= 2
        k *= 2
    psel, pbset, ptt, pax = pend
    p = jnp.where(psel, _partner(p, pbset, ptt, pax), p)

    w_s = lax.bitcast_convert_type(p & jnp.int32(-65536), jnp.float32)
    e_s = lax.bitcast_convert_type(p << 16, jnp.float32)

    tri = (ri <= ci).astype(jnp.float32)
    ps = jnp.dot(w_s, tri, preferred_element_type=jnp.float32)
    rs = ps[:, _L - 1:_L]
    ro = jnp.dot((ri >= ci).astype(jnp.float32), rs,
                 preferred_element_type=jnp.float32)
    prefix = ps + (ro - rs)
    risk = mx + jnp.log(prefix)
    num = jnp.sum(e_s * risk) - elh
    out_ref[...] = (num / den).reshape(1, 1)


@jax.jit
def kernel(log_h, y_gts):
    d = y_gts[:, 0]
    e = y_gts[:, 1]
    out = pl.pallas_call(
        _cox_body,
        out_shape=jax.ShapeDtypeStruct((1, 1), jnp.float32),
    )(d.reshape(_R, _L), log_h.reshape(_R, _L), e.reshape(_R, _L))
    return out[0, 0]
